# baseline (speedup 1.0000x reference)
import functools

import jax
import jax.numpy as jnp
from jax import lax
from jax.experimental import pallas as pl
from jax.experimental.pallas import tpu as pltpu
from jax.experimental.pallas import tpu_sc as plsc

_B, _T = 4096, 50
_DIMS = (64, 64, 32, 32)
_OFFS = (0, 64, 128, 160)
_DSUM = 192
_NC, _NS = 2, 16
_NW = _NC * _NS
_BLK = _B // _NW
_L = 16

_mesh = plsc.VectorSubcoreMesh(core_axis_name="c", subcore_axis_name="s")


@functools.partial(
    pl.kernel,
    out_type=jax.ShapeDtypeStruct((_T, _DSUM, _B), jnp.float32),
    mesh=_mesh,
    compiler_params=pltpu.CompilerParams(
        use_tc_tiling_on_sc=False, needs_layout_passes=False),
    scratch_types=[
        pltpu.VMEM((_T, _BLK), jnp.int32),
        pltpu.VMEM((_T, _BLK), jnp.int32),
        pltpu.VMEM((_T, _BLK), jnp.int32),
        pltpu.VMEM((_T, _BLK), jnp.int32),
        pltpu.VMEM((2, _BLK, 64), jnp.float32),
        pltpu.VMEM((2, _BLK, 64), jnp.float32),
        pltpu.VMEM((2, _BLK, 32), jnp.float32),
        pltpu.VMEM((2, _BLK, 32), jnp.float32),
        pltpu.VMEM((2, _DSUM, _BLK), jnp.float32),
        pltpu.SemaphoreType.DMA,
        pltpu.SemaphoreType.DMA,
        pltpu.SemaphoreType.DMA,
        pltpu.SemaphoreType.DMA,
    ],
)
def _emb_gather(x0_h, x1_h, x2_h, x3_h, w0_h, w1_h, w2_h, w3_h, out_h,
                i0, i1, i2, i3, s0, s1, s2, s3, ot, gsem0, gsem1, osem0, osem1):
    wid = lax.axis_index("s") * _NC + lax.axis_index("c")
    b0 = wid * _BLK

    pltpu.sync_copy(x0_h.at[:, pl.ds(b0, _BLK)], i0)
    pltpu.sync_copy(x1_h.at[:, pl.ds(b0, _BLK)], i1)
    pltpu.sync_copy(x2_h.at[:, pl.ds(b0, _BLK)], i2)
    pltpu.sync_copy(x3_h.at[:, pl.ds(b0, _BLK)], i3)

    idx_refs = (i0, i1, i2, i3)
    w_refs = (w0_h, w1_h, w2_h, w3_h)
    stages = (s0, s1, s2, s3)
    gsems = (gsem0, gsem1)
    osems = (osem0, osem1)

    def gather_copies(t, sl):
        for k in range(4):
            src = w_refs[k].at[idx_refs[k].at[t]]
            yield src, stages[k].at[sl], gsems[sl]

    def out_copies(t, sl):
        yield ot.at[sl], out_h.at[t, :, pl.ds(b0, _BLK)], osems[sl]

    def fire(copies):
        for src, dst, sem in copies:
            pltpu.async_copy(src, dst, sem)

    def drain(copies):
        for src, dst, sem in copies:
            pltpu.make_async_copy(src, dst, sem).wait()

    lane = lax.iota(jnp.int32, _L)

    def transpose_into(sl):
        def bq_body(bq):
            rows = bq * _L + lane
            for k in range(4):
                for c in range(_DIMS[k]):
                    cols = jnp.full((_L,), c, jnp.int32)
                    v = plsc.load_gather(stages[k].at[sl], [rows, cols])
                    ot[sl, _OFFS[k] + c, pl.ds(bq * _L, _L)] = v

        pl.loop(0, _BLK // _L)(bq_body)

    fire(gather_copies(0, 0))

    def pair_body(t):
        for sl in range(2):
            ct = t + sl

            @pl.when(ct >= 1)
            def _():
                drain(out_copies(ct - 1, 1 - sl))

            @pl.when(ct + 1 <= _T - 1)
            def _():
                fire(gather_copies(ct + 1, 1 - sl))

            drain(gather_copies(ct, sl))
            transpose_into(sl)
            fire(out_copies(ct, sl))

    pl.loop(0, _T, step=2)(pair_body)
    drain(out_copies(_T - 1, 1))


def kernel(x0, x1, x2, x3, W0, W1, W2, W3):
    xs = [x.astype(jnp.int32).T for x in (x0, x1, x2, x3)]
    out_t = _emb_gather(xs[0], xs[1], xs[2], xs[3], W0, W1, W2, W3)
    return jnp.transpose(out_t, (2, 0, 1))


# speedup vs baseline: 2.1680x; 2.1680x over previous
import functools

import jax
import jax.numpy as jnp
from jax import lax
from jax.experimental import pallas as pl
from jax.experimental.pallas import tpu as pltpu
from jax.experimental.pallas import tpu_sc as plsc

_B, _T = 4096, 50
_TP = 64
_DIMS = (64, 64, 32, 32)
_OFFS = (0, 64, 128, 160)
_DSUM = 192
_NC, _NS = 2, 16
_NW = _NC * _NS
_BPW = _B // _NW
_L = 16

_mesh = plsc.VectorSubcoreMesh(core_axis_name="c", subcore_axis_name="s")


@functools.partial(
    pl.kernel,
    out_type=jax.ShapeDtypeStruct((_B, _T, _DSUM), jnp.float32),
    mesh=_mesh,
    scratch_types=[
        pltpu.VMEM((_BPW * _TP,), jnp.int32),
        pltpu.VMEM((_BPW * _TP,), jnp.int32),
        pltpu.VMEM((_BPW * _TP,), jnp.int32),
        pltpu.VMEM((_BPW * _TP,), jnp.int32),
        pltpu.VMEM((2, _T, _DSUM), jnp.float32),
        pltpu.VMEM((2, _T, 128), jnp.float32),
        pltpu.VMEM((2, _T, 128), jnp.float32),
        pltpu.VMEM((2, _T, 128), jnp.float32),
        pltpu.SemaphoreType.DMA,
        pltpu.SemaphoreType.DMA,
        pltpu.SemaphoreType.DMA,
        pltpu.SemaphoreType.DMA,
    ],
)
def _emb_gather(x0_h, x1_h, x2_h, x3_h, w0_h, w1_h, w2_h, w3_h, out_h,
                i0, i1, i2, i3, so, s1, s2, s3, gsem0, gsem1, osem0, osem1):
    wid = lax.axis_index("s") * _NC + lax.axis_index("c")
    base = wid * _BPW * _TP
    bbase = wid * _BPW

    pltpu.sync_copy(x0_h.at[pl.ds(base, _BPW * _TP)], i0)
    pltpu.sync_copy(x1_h.at[pl.ds(base, _BPW * _TP)], i1)
    pltpu.sync_copy(x2_h.at[pl.ds(base, _BPW * _TP)], i2)
    pltpu.sync_copy(x3_h.at[pl.ds(base, _BPW * _TP)], i3)

    idx_refs = (i0, i1, i2, i3)
    w_refs = (w0_h, w1_h, w2_h, w3_h)
    side = (s1, s2, s3)
    gsems = (gsem0, gsem1)
    osems = (osem0, osem1)

    def gather_copies(j, sl):
        off = j * _TP
        srcs = [w_refs[k].at[idx_refs[k].at[pl.ds(off, _T)]] for k in range(4)]
        yield srcs[0], so.at[sl, :, pl.ds(0, 128)], gsems[sl]
        for k in range(1, 4):
            yield srcs[k], side[k - 1].at[sl], gsems[sl]

    def out_copies(j, sl):
        yield so.at[sl], out_h.at[bbase + j], osems[sl]

    def fire(copies):
        for src, dst, sem in copies:
            pltpu.async_copy(src, dst, sem)

    def drain(copies):
        for src, dst, sem in copies:
            pltpu.make_async_copy(src, dst, sem).wait()

    def assemble(sl):
        for t in range(_T):
            for k in range(1, 4):
                sref = side[k - 1]
                for c in range(0, _DIMS[k], _L):
                    so[sl, t, pl.ds(_OFFS[k] + c, _L)] = sref[sl, t, pl.ds(c, _L)]

    fire(gather_copies(0, 0))

    def pair_body(j):
        for sl in range(2):
            cj = j + sl

            @pl.when(cj >= 1)
            def _():
                drain(out_copies(cj - 1, 1 - sl))

            @pl.when(cj + 1 <= _BPW - 1)
            def _():
                fire(gather_copies(cj + 1, 1 - sl))

            drain(gather_copies(cj, sl))
            assemble(sl)
            fire(out_copies(cj, sl))

    pl.loop(0, _BPW, step=2)(pair_body)
    drain(out_copies(_BPW - 1, 1))


def kernel(x0, x1, x2, x3, W0, W1, W2, W3):
    xs = [
        jnp.pad(x.astype(jnp.int32), ((0, 0), (0, _TP - _T))).reshape(-1)
        for x in (x0, x1, x2, x3)
    ]
    ws = [jnp.pad(w, ((0, 0), (0, 128 - w.shape[1]))) for w in (W0, W1, W2, W3)]
    return _emb_gather(xs[0], xs[1], xs[2], xs[3], ws[0], ws[1], ws[2], ws[3])
